# baseline (speedup 1.0000x reference)
import functools

import jax
import jax.numpy as jnp
from jax import lax
from jax.experimental import pallas as pl
from jax.experimental.pallas import tpu as pltpu
from jax.experimental.pallas import tpu_sc as plsc

ROWS = 2048
COLS = 8192
NUM_CORES = 2
NUM_SUBCORES = 16
NW = NUM_CORES * NUM_SUBCORES
ROWS_W = ROWS // NW
BLK_R = 8
BLK_C = 2048
N_BLK = (ROWS_W // BLK_R) * (COLS // BLK_C)
CB = COLS // BLK_C
G = N_BLK // 2
LANES = 16

_GDN = lax.GatherDimensionNumbers(
    offset_dims=(), collapsed_slice_dims=(0,), start_index_map=(0,))


def _vgather(v, idx):
    return lax.gather(v, idx[:, None], _GDN, slice_sizes=(1,),
                      mode=lax.GatherScatterMode.PROMISE_IN_BOUNDS)


_mesh = plsc.VectorSubcoreMesh(core_axis_name="c", subcore_axis_name="s")


@functools.partial(
    pl.kernel,
    mesh=_mesh,
    out_type=jax.ShapeDtypeStruct((ROWS, COLS), jnp.float32),
    scratch_types=[
        pltpu.VMEM((BLK_R, BLK_C), jnp.int32),
        pltpu.VMEM((BLK_R, BLK_C), jnp.int32),
        pltpu.VMEM((BLK_R, BLK_C), jnp.int32),
        pltpu.VMEM((BLK_R, BLK_C), jnp.int32),
        pltpu.SemaphoreType.DMA,
        pltpu.SemaphoreType.DMA,
        pltpu.SemaphoreType.DMA,
        pltpu.SemaphoreType.DMA,
    ],
)
def _sc_prune(x_f, out_f, in0, in1, out0, out1,
              sem_in0, sem_in1, sem_out0, sem_out1):
    x_hbm = x_f.bitcast(jnp.int32)
    out_hbm = out_f.bitcast(jnp.int32)
    wid = lax.axis_index("s") * NUM_CORES + lax.axis_index("c")
    row0 = wid * ROWS_W

    lane = lax.iota(jnp.int32, LANES)
    p1 = lane ^ 1
    p2 = lane ^ 2
    p3 = lane ^ 3
    t1 = lane & 1
    t2 = (lane >> 1) & 1
    t3 = t2

    def blk(ci):
        r = row0 + (ci // CB) * BLK_R
        c = (ci % CB) * BLK_C
        return r, c

    def start_in(ci, buf, sem):
        r, c = blk(ci)
        pltpu.async_copy(x_hbm.at[pl.ds(r, BLK_R), pl.ds(c, BLK_C)], buf, sem)

    def wait_in(buf, sem):
        pltpu.make_async_copy(
            x_hbm.at[pl.ds(row0, BLK_R), pl.ds(0, BLK_C)], buf, sem).wait()

    def start_out(ci, buf, sem):
        r, c = blk(ci)
        pltpu.async_copy(buf, out_hbm.at[pl.ds(r, BLK_R), pl.ds(c, BLK_C)], sem)

    def wait_out(buf, sem):
        pltpu.make_async_copy(
            buf, out_hbm.at[pl.ds(row0, BLK_R), pl.ds(0, BLK_C)], sem).wait()

    def compute(in_v, out_v):
        def _row(r):
            def _body(it, carry):
                x = in_v[r, pl.ds(it * LANES, LANES)]
                ab = x & 0x7FFFFFFF
                b1 = _vgather(ab, p1)
                b2 = _vgather(ab, p2)
                b3 = _vgather(ab, p3)
                c1 = (b1 - t1) < ab
                c2 = (b2 - t2) < ab
                c3 = (b3 - t3) < ab
                keep = (c1 & c2) | ((c1 | c2) & c3)
                out_v[r, pl.ds(it * LANES, LANES)] = jnp.where(keep, x, 0)
                return carry

            lax.fori_loop(0, BLK_C // LANES, _body, 0, unroll=16)

        for r in range(BLK_R):
            _row(r)

    start_in(0, in0, sem_in0)

    def pair_body(g, carry):
        c0 = 2 * g
        start_in(c0 + 1, in1, sem_in1)
        wait_in(in0, sem_in0)

        @pl.when(g > 0)
        def _():
            wait_out(out0, sem_out0)

        compute(in0, out0)
        start_out(c0, out0, sem_out0)

        @pl.when(g < G - 1)
        def _():
            start_in(c0 + 2, in0, sem_in0)

        wait_in(in1, sem_in1)

        @pl.when(g > 0)
        def _():
            wait_out(out1, sem_out1)

        compute(in1, out1)
        start_out(c0 + 1, out1, sem_out1)
        return carry

    lax.fori_loop(0, G, pair_body, 0)
    wait_out(out0, sem_out0)
    wait_out(out1, sem_out1)


def kernel(inputs):
    return _sc_prune(inputs)


# speedup vs baseline: 1.5542x; 1.5542x over previous
import functools

import jax
import jax.numpy as jnp
from jax import lax
from jax.experimental import pallas as pl
from jax.experimental.pallas import tpu as pltpu
from jax.experimental.pallas import tpu_sc as plsc

ROWS = 2048
COLS = 8192
NUM_CORES = 2
NUM_SUBCORES = 16
NW = NUM_CORES * NUM_SUBCORES
ROWS_W = ROWS // NW
BLK_R = 8
BLK_C = 2048
N_BLK = (ROWS_W // BLK_R) * (COLS // BLK_C)
CB = COLS // BLK_C
G = N_BLK // 2
LANES = 16

_GDN = lax.GatherDimensionNumbers(
    offset_dims=(), collapsed_slice_dims=(0,), start_index_map=(0,))


def _vgather(v, idx):
    return lax.gather(v, idx[:, None], _GDN, slice_sizes=(1,),
                      mode=lax.GatherScatterMode.PROMISE_IN_BOUNDS)


_mesh = plsc.VectorSubcoreMesh(core_axis_name="c", subcore_axis_name="s")


@functools.partial(
    pl.kernel,
    mesh=_mesh,
    out_type=jax.ShapeDtypeStruct((ROWS, COLS), jnp.float32),
    scratch_types=[
        pltpu.VMEM((BLK_R, BLK_C), jnp.int32),
        pltpu.VMEM((BLK_R, BLK_C), jnp.int32),
        pltpu.VMEM((BLK_R, BLK_C), jnp.int32),
        pltpu.VMEM((BLK_R, BLK_C), jnp.int32),
        pltpu.SemaphoreType.DMA,
        pltpu.SemaphoreType.DMA,
        pltpu.SemaphoreType.DMA,
        pltpu.SemaphoreType.DMA,
    ],
)
def _sc_prune(x_f, out_f, in0, in1, out0, out1,
              sem_in0, sem_in1, sem_out0, sem_out1):
    x_hbm = x_f.bitcast(jnp.int32)
    out_hbm = out_f.bitcast(jnp.int32)
    wid = lax.axis_index("s") * NUM_CORES + lax.axis_index("c")
    row0 = wid * ROWS_W

    lane = lax.iota(jnp.int32, LANES)
    p1 = lane ^ 1
    p2 = lane ^ 2
    p3 = lane ^ 3
    t1 = lane & 1
    t2 = (lane >> 1) & 1
    t3 = t2

    def blk(ci):
        r = row0 + (ci // CB) * BLK_R
        c = (ci % CB) * BLK_C
        return r, c

    def start_in(ci, buf, sem):
        r, c = blk(ci)
        pltpu.async_copy(x_hbm.at[pl.ds(r, BLK_R), pl.ds(c, BLK_C)], buf, sem)

    def wait_in(buf, sem):
        pltpu.make_async_copy(
            x_hbm.at[pl.ds(row0, BLK_R), pl.ds(0, BLK_C)], buf, sem).wait()

    def start_out(ci, buf, sem):
        r, c = blk(ci)
        pltpu.async_copy(buf, out_hbm.at[pl.ds(r, BLK_R), pl.ds(c, BLK_C)], sem)

    def wait_out(buf, sem):
        pltpu.make_async_copy(
            buf, out_hbm.at[pl.ds(row0, BLK_R), pl.ds(0, BLK_C)], sem).wait()

    def compute(in_v, out_v):
        def _row(r):
            def _body(it, carry):
                x = in_v[r, pl.ds(it * LANES, LANES)]
                ab = x & 0x7FFFFFFF
                b1 = _vgather(ab, p1)
                b2 = _vgather(ab, p2)
                b3 = _vgather(ab, p3)
                c1 = (b1 - t1) < ab
                c2 = (b2 - t2) < ab
                c3 = (b3 - t3) < ab
                keep = (c1 & c2) | ((c1 | c2) & c3)
                out_v[r, pl.ds(it * LANES, LANES)] = jnp.where(keep, x, 0)
                return carry

            lax.fori_loop(0, BLK_C // LANES, _body, 0, unroll=4)

        for r in range(BLK_R):
            _row(r)

    start_in(0, in0, sem_in0)

    def pair_body(g, carry):
        c0 = 2 * g
        start_in(c0 + 1, in1, sem_in1)
        wait_in(in0, sem_in0)

        @pl.when(g > 0)
        def _():
            wait_out(out0, sem_out0)

        compute(in0, out0)
        start_out(c0, out0, sem_out0)

        @pl.when(g < G - 1)
        def _():
            start_in(c0 + 2, in0, sem_in0)

        wait_in(in1, sem_in1)

        @pl.when(g > 0)
        def _():
            wait_out(out1, sem_out1)

        compute(in1, out1)
        start_out(c0 + 1, out1, sem_out1)
        return carry

    lax.fori_loop(0, G, pair_body, 0)
    wait_out(out0, sem_out0)
    wait_out(out1, sem_out1)


def kernel(inputs):
    return _sc_prune(inputs)


# speedup vs baseline: 2.0209x; 1.3003x over previous
import functools

import jax
import jax.numpy as jnp
from jax import lax
from jax.experimental import pallas as pl
from jax.experimental.pallas import tpu as pltpu
from jax.experimental.pallas import tpu_sc as plsc

ROWS = 2048
COLS = 8192
NUM_CORES = 2
NUM_SUBCORES = 16
NW = NUM_CORES * NUM_SUBCORES
ROWS_W = ROWS // NW
BLK_R = 8
BLK_C = 2048
N_BLK = (ROWS_W // BLK_R) * (COLS // BLK_C)
CB = COLS // BLK_C
G = N_BLK // 2
LANES = 16

_GDN = lax.GatherDimensionNumbers(
    offset_dims=(), collapsed_slice_dims=(0,), start_index_map=(0,))


def _vgather(v, idx):
    return lax.gather(v, idx[:, None], _GDN, slice_sizes=(1,),
                      mode=lax.GatherScatterMode.PROMISE_IN_BOUNDS)


_mesh = plsc.VectorSubcoreMesh(core_axis_name="c", subcore_axis_name="s")


@functools.partial(
    pl.kernel,
    mesh=_mesh,
    out_type=jax.ShapeDtypeStruct((ROWS, COLS), jnp.float32),
    scratch_types=[
        pltpu.VMEM((BLK_R, BLK_C), jnp.int32),
        pltpu.VMEM((BLK_R, BLK_C), jnp.int32),
        pltpu.VMEM((BLK_R, BLK_C), jnp.int32),
        pltpu.VMEM((BLK_R, BLK_C), jnp.int32),
        pltpu.SemaphoreType.DMA,
        pltpu.SemaphoreType.DMA,
        pltpu.SemaphoreType.DMA,
        pltpu.SemaphoreType.DMA,
    ],
)
def _sc_prune(x_f, out_f, in0, in1, out0, out1,
              sem_in0, sem_in1, sem_out0, sem_out1):
    x_hbm = x_f.bitcast(jnp.int32)
    out_hbm = out_f.bitcast(jnp.int32)
    wid = lax.axis_index("s") * NUM_CORES + lax.axis_index("c")
    row0 = wid * ROWS_W

    lane = lax.iota(jnp.int32, LANES)
    p1 = lane ^ 1
    p2 = lane ^ 2
    p3 = lane ^ 3
    t1 = lane & 1
    t2 = (lane >> 1) & 1
    t3 = t2

    def blk(ci):
        r = row0 + (ci // CB) * BLK_R
        c = (ci % CB) * BLK_C
        return r, c

    def start_in(ci, buf, sem):
        r, c = blk(ci)
        pltpu.async_copy(x_hbm.at[pl.ds(r, BLK_R), pl.ds(c, BLK_C)], buf, sem)

    def wait_in(buf, sem):
        pltpu.make_async_copy(
            x_hbm.at[pl.ds(row0, BLK_R), pl.ds(0, BLK_C)], buf, sem).wait()

    def start_out(ci, buf, sem):
        r, c = blk(ci)
        pltpu.async_copy(buf, out_hbm.at[pl.ds(r, BLK_R), pl.ds(c, BLK_C)], sem)

    def wait_out(buf, sem):
        pltpu.make_async_copy(
            buf, out_hbm.at[pl.ds(row0, BLK_R), pl.ds(0, BLK_C)], sem).wait()

    def compute(in_v, out_v):
        def _row(r):
            def _body(it, carry):
                x = in_v[r, pl.ds(it * LANES, LANES)]
                ab = x & 0x7FFFFFFF
                b1 = _vgather(ab, p1)
                b2 = _vgather(ab, p2)
                b3 = _vgather(ab, p3)
                c1 = (b1 - t1) < ab
                c2 = (b2 - t2) < ab
                c3 = (b3 - t3) < ab
                keep = (c1 & c2) | ((c1 | c2) & c3)
                out_v[r, pl.ds(it * LANES, LANES)] = jnp.where(keep, x, 0)
                return carry

            lax.fori_loop(0, BLK_C // LANES, _body, 0, unroll=2)

        for r in range(BLK_R):
            _row(r)

    start_in(0, in0, sem_in0)

    def pair_body(g, carry):
        c0 = 2 * g
        start_in(c0 + 1, in1, sem_in1)
        wait_in(in0, sem_in0)

        @pl.when(g > 0)
        def _():
            wait_out(out0, sem_out0)

        compute(in0, out0)
        start_out(c0, out0, sem_out0)

        @pl.when(g < G - 1)
        def _():
            start_in(c0 + 2, in0, sem_in0)

        wait_in(in1, sem_in1)

        @pl.when(g > 0)
        def _():
            wait_out(out1, sem_out1)

        compute(in1, out1)
        start_out(c0 + 1, out1, sem_out1)
        return carry

    lax.fori_loop(0, G, pair_body, 0)
    wait_out(out0, sem_out0)
    wait_out(out1, sem_out1)


def kernel(inputs):
    return _sc_prune(inputs)


# speedup vs baseline: 2.4411x; 1.2079x over previous
import functools

import jax
import jax.numpy as jnp
from jax import lax
from jax.experimental import pallas as pl
from jax.experimental.pallas import tpu as pltpu
from jax.experimental.pallas import tpu_sc as plsc

ROWS = 2048
COLS = 8192
NUM_CORES = 2
NUM_SUBCORES = 16
NW = NUM_CORES * NUM_SUBCORES
ROWS_W = ROWS // NW
BLK_R = 8
BLK_C = 2048
N_BLK = (ROWS_W // BLK_R) * (COLS // BLK_C)
CB = COLS // BLK_C
G = N_BLK // 2
LANES = 16

_GDN = lax.GatherDimensionNumbers(
    offset_dims=(), collapsed_slice_dims=(0,), start_index_map=(0,))


def _vgather(v, idx):
    return lax.gather(v, idx[:, None], _GDN, slice_sizes=(1,),
                      mode=lax.GatherScatterMode.PROMISE_IN_BOUNDS)


_mesh = plsc.VectorSubcoreMesh(core_axis_name="c", subcore_axis_name="s")


@functools.partial(
    pl.kernel,
    mesh=_mesh,
    out_type=jax.ShapeDtypeStruct((ROWS, COLS), jnp.float32),
    scratch_types=[
        pltpu.VMEM((BLK_R, BLK_C), jnp.int32),
        pltpu.VMEM((BLK_R, BLK_C), jnp.int32),
        pltpu.VMEM((BLK_R, BLK_C), jnp.int32),
        pltpu.VMEM((BLK_R, BLK_C), jnp.int32),
        pltpu.SemaphoreType.DMA,
        pltpu.SemaphoreType.DMA,
        pltpu.SemaphoreType.DMA,
        pltpu.SemaphoreType.DMA,
    ],
)
def _sc_prune(x_f, out_f, in0, in1, out0, out1,
              sem_in0, sem_in1, sem_out0, sem_out1):
    x_hbm = x_f.bitcast(jnp.int32)
    out_hbm = out_f.bitcast(jnp.int32)
    wid = lax.axis_index("s") * NUM_CORES + lax.axis_index("c")
    row0 = wid * ROWS_W

    lane = lax.iota(jnp.int32, LANES)
    p1 = lane ^ 1
    p2 = lane ^ 2
    p3 = lane ^ 3
    t1 = lane & 1
    t2 = (lane >> 1) & 1
    t3 = t2

    def blk(ci):
        r = row0 + (ci // CB) * BLK_R
        c = (ci % CB) * BLK_C
        return r, c

    def start_in(ci, buf, sem):
        r, c = blk(ci)
        pltpu.async_copy(x_hbm.at[pl.ds(r, BLK_R), pl.ds(c, BLK_C)], buf, sem)

    def wait_in(buf, sem):
        pltpu.make_async_copy(
            x_hbm.at[pl.ds(row0, BLK_R), pl.ds(0, BLK_C)], buf, sem).wait()

    def start_out(ci, buf, sem):
        r, c = blk(ci)
        pltpu.async_copy(buf, out_hbm.at[pl.ds(r, BLK_R), pl.ds(c, BLK_C)], sem)

    def wait_out(buf, sem):
        pltpu.make_async_copy(
            buf, out_hbm.at[pl.ds(row0, BLK_R), pl.ds(0, BLK_C)], sem).wait()

    def compute(in_v, out_v):
        def _row(r):
            def _body(it, carry):
                x = in_v[r, pl.ds(it * LANES, LANES)]
                ab = x & 0x7FFFFFFF
                b1 = _vgather(ab, p1)
                b2 = _vgather(ab, p2)
                b3 = _vgather(ab, p3)
                c1 = (b1 - t1) < ab
                c2 = (b2 - t2) < ab
                c3 = (b3 - t3) < ab
                keep = (c1 & c2) | ((c1 | c2) & c3)
                out_v[r, pl.ds(it * LANES, LANES)] = jnp.where(keep, x, 0)
                return carry

            lax.fori_loop(0, BLK_C // LANES, _body, 0)

        for r in range(BLK_R):
            _row(r)

    start_in(0, in0, sem_in0)

    def pair_body(g, carry):
        c0 = 2 * g
        start_in(c0 + 1, in1, sem_in1)
        wait_in(in0, sem_in0)

        @pl.when(g > 0)
        def _():
            wait_out(out0, sem_out0)

        compute(in0, out0)
        start_out(c0, out0, sem_out0)

        @pl.when(g < G - 1)
        def _():
            start_in(c0 + 2, in0, sem_in0)

        wait_in(in1, sem_in1)

        @pl.when(g > 0)
        def _():
            wait_out(out1, sem_out1)

        compute(in1, out1)
        start_out(c0 + 1, out1, sem_out1)
        return carry

    lax.fori_loop(0, G, pair_body, 0)
    wait_out(out0, sem_out0)
    wait_out(out1, sem_out1)


def kernel(inputs):
    return _sc_prune(inputs)


# speedup vs baseline: 2.4548x; 1.0056x over previous
import functools

import jax
import jax.numpy as jnp
from jax import lax
from jax.experimental import pallas as pl
from jax.experimental.pallas import tpu as pltpu
from jax.experimental.pallas import tpu_sc as plsc

ROWS = 2048
COLS = 8192
NUM_CORES = 2
NUM_SUBCORES = 16
NW = NUM_CORES * NUM_SUBCORES
ROWS_W = ROWS // NW
BLK_R = 8
BLK_C = 2048
N_BLK = (ROWS_W // BLK_R) * (COLS // BLK_C)
CB = COLS // BLK_C
G = N_BLK // 2
LANES = 16

_GDN = lax.GatherDimensionNumbers(
    offset_dims=(), collapsed_slice_dims=(0,), start_index_map=(0,))


def _vgather(v, idx):
    return lax.gather(v, idx[:, None], _GDN, slice_sizes=(1,),
                      mode=lax.GatherScatterMode.PROMISE_IN_BOUNDS)


_mesh = plsc.VectorSubcoreMesh(core_axis_name="c", subcore_axis_name="s")


@functools.partial(
    pl.kernel,
    mesh=_mesh,
    out_type=jax.ShapeDtypeStruct((ROWS, COLS), jnp.float32),
    scratch_types=[
        pltpu.VMEM((BLK_R, BLK_C), jnp.int32),
        pltpu.VMEM((BLK_R, BLK_C), jnp.int32),
        pltpu.VMEM((BLK_R, BLK_C), jnp.int32),
        pltpu.VMEM((BLK_R, BLK_C), jnp.int32),
        pltpu.SemaphoreType.DMA,
        pltpu.SemaphoreType.DMA,
        pltpu.SemaphoreType.DMA,
        pltpu.SemaphoreType.DMA,
    ],
)
def _sc_prune(x_f, out_f, in0, in1, out0, out1,
              sem_in0, sem_in1, sem_out0, sem_out1):
    x_hbm = x_f.bitcast(jnp.int32)
    out_hbm = out_f.bitcast(jnp.int32)
    wid = lax.axis_index("s") * NUM_CORES + lax.axis_index("c")
    row0 = wid * ROWS_W

    lane = lax.iota(jnp.int32, LANES)
    p1 = lane ^ 1
    p2 = lane ^ 2
    p3 = lane ^ 3
    t1 = lane & 1
    t2 = (lane >> 1) & 1
    t3 = t2

    def blk(ci):
        r = row0 + (ci // CB) * BLK_R
        c = (ci % CB) * BLK_C
        return r, c

    def start_in(ci, buf, sem):
        r, c = blk(ci)
        pltpu.async_copy(x_hbm.at[pl.ds(r, BLK_R), pl.ds(c, BLK_C)], buf, sem)

    def wait_in(buf, sem):
        pltpu.make_async_copy(
            x_hbm.at[pl.ds(row0, BLK_R), pl.ds(0, BLK_C)], buf, sem).wait()

    def start_out(ci, buf, sem):
        r, c = blk(ci)
        pltpu.async_copy(buf, out_hbm.at[pl.ds(r, BLK_R), pl.ds(c, BLK_C)], sem)

    def wait_out(buf, sem):
        pltpu.make_async_copy(
            buf, out_hbm.at[pl.ds(row0, BLK_R), pl.ds(0, BLK_C)], sem).wait()

    def compute(in_v, out_v):
        def _row(r):
            def _body(it, carry):
                x = in_v[r, pl.ds(it * LANES, LANES)]
                ab = x & 0x7FFFFFFF
                a1 = ab + t1
                a2 = ab + t2
                b1 = _vgather(ab, p1)
                b2 = _vgather(ab, p2)
                b3 = _vgather(ab, p3)
                c1 = b1 < a1
                c2 = b2 < a2
                c3 = b3 < a2
                keep = (c1 & c2) | ((c1 | c2) & c3)
                out_v[r, pl.ds(it * LANES, LANES)] = jnp.where(keep, x, 0)
                return carry

            lax.fori_loop(0, BLK_C // LANES, _body, 0)

        for r in range(BLK_R):
            _row(r)

    start_in(0, in0, sem_in0)

    def pair_body(g, carry):
        c0 = 2 * g
        start_in(c0 + 1, in1, sem_in1)
        wait_in(in0, sem_in0)

        @pl.when(g > 0)
        def _():
            wait_out(out0, sem_out0)

        compute(in0, out0)
        start_out(c0, out0, sem_out0)

        @pl.when(g < G - 1)
        def _():
            start_in(c0 + 2, in0, sem_in0)

        wait_in(in1, sem_in1)

        @pl.when(g > 0)
        def _():
            wait_out(out1, sem_out1)

        compute(in1, out1)
        start_out(c0 + 1, out1, sem_out1)
        return carry

    lax.fori_loop(0, G, pair_body, 0)
    wait_out(out0, sem_out0)
    wait_out(out1, sem_out1)


def kernel(inputs):
    return _sc_prune(inputs)
